# baseline (speedup 1.0000x reference)
import jax
import jax.numpy as jnp
from jax.experimental import pallas as pl

_SCALE = 20.0
_LOG2E = 1.4426950408889634
_LN2 = 0.6931471805599453
_BT = 512
_CT = 2560


def _ce_kernel(x_ref, t_ref, lut_ref, ones_ref, o_ref, *, chunks, bt,
               batch, n_classes):
    i = pl.program_id(0)
    x = x_ref[...]
    t = t_ref[...]
    ones = ones_ref[...]
    colf = jax.lax.broadcasted_iota(jnp.int32, (bt, chunks[0][1]), 1)
    m = jnp.full((bt, 1), -jnp.inf, jnp.float32)
    s = jnp.zeros((bt, 1), jnp.float32)
    tg = jnp.zeros((bt, 1), jnp.float32)
    for c0, csz in chunks:
        lut_blk = lut_ref[c0:c0 + csz, :]
        l2 = jax.lax.dot_general(
            x, lut_blk, (((1,), (1,)), ((), ())),
            preferred_element_type=jnp.float32).astype(jnp.bfloat16)
        col = colf[:, :csz]
        if c0 + csz > n_classes:
            l2 = jnp.where(col < n_classes - c0, l2,
                           jnp.bfloat16(-jnp.inf))
        cmax = jnp.max(l2, axis=1, keepdims=True).astype(jnp.float32)
        mn = jnp.maximum(m, cmax)
        e = jnp.exp2(l2 - mn.astype(jnp.bfloat16))
        csum = jax.lax.dot_general(
            e, ones[:csz, :], (((1,), (0,)), ((), ())),
            preferred_element_type=jnp.float32)[:, :1]
        s = s * jnp.exp2(m - mn) + csum
        m = mn
        tg = tg + jnp.sum(
            jnp.where(col == (t - c0), l2, jnp.bfloat16(0)),
            axis=1, keepdims=True).astype(jnp.float32)
    part = (jnp.sum(m + jnp.log2(s) - tg) * (_LN2 / batch)).reshape(1, 1)

    @pl.when(i == 0)
    def _init():
        o_ref[...] = jnp.zeros((1, 1), jnp.float32)

    o_ref[...] += part


def kernel(inputs, targets, lut_ccc, lut_icc):
    b, f = inputs.shape
    n_classes = lut_icc.shape[0]
    bt = _BT if b % _BT == 0 else b
    chunks = []
    c0 = 0
    while c0 < n_classes:
        csz = min(_CT, ((n_classes - c0 + 7) // 8) * 8)
        chunks.append((c0, csz))
        c0 += csz
    cp = c0
    lut_pad = lut_icc if cp == n_classes else jnp.pad(
        lut_icc, ((0, cp - n_classes), (0, 0)))
    x16 = (inputs * (_SCALE * _LOG2E)).astype(jnp.bfloat16)
    lut16 = lut_pad.astype(jnp.bfloat16)
    ones = jnp.ones((chunks[0][1], 128), jnp.bfloat16)
    t2 = targets.reshape(b, 1)
    out = pl.pallas_call(
        lambda xr, tr, lr, onr, orf: _ce_kernel(
            xr, tr, lr, onr, orf, chunks=chunks, bt=bt, batch=b, n_classes=n_classes),
        grid=(b // bt,),
        in_specs=[
            pl.BlockSpec((bt, f), lambda i: (i, 0)),
            pl.BlockSpec((bt, 1), lambda i: (i, 0)),
            pl.BlockSpec((cp, f), lambda i: (0, 0)),
            pl.BlockSpec((chunks[0][1], 128), lambda i: (0, 0)),
        ],
        out_specs=pl.BlockSpec((1, 1), lambda i: (0, 0)),
        out_shape=jax.ShapeDtypeStruct((1, 1), jnp.float32),
    )(x16, t2, lut16, ones)
    loss = out[0, 0]
    return (loss, lut_icc, lut_icc)


# speedup vs baseline: 1.1991x; 1.1991x over previous
import jax
import jax.numpy as jnp
from jax.experimental import pallas as pl

_SCALE = 20.0
_LOG2E = 1.4426950408889634
_LN2 = 0.6931471805599453
_BT = 512
_CT = 2560


def _ce_kernel(x_ref, t_ref, lut_ref, ones_ref, o_ref, *, chunks, bt,
               batch, n_classes):
    i = pl.program_id(0)
    x = x_ref[...]
    t = t_ref[...]
    ones = ones_ref[...]
    colf = jax.lax.broadcasted_iota(jnp.int32, (bt, chunks[0][1]), 1)
    m = jnp.full((bt, 1), -jnp.inf, jnp.float32)
    s = jnp.zeros((bt, 1), jnp.float32)
    tg = jnp.zeros((bt, 1), jnp.float32)
    for c0, csz in chunks:
        lut_blk = lut_ref[c0:c0 + csz, :]
        l2 = jax.lax.dot_general(
            x, lut_blk, (((1,), (1,)), ((), ())),
            preferred_element_type=jnp.float32).astype(jnp.bfloat16)
        col = colf[:, :csz]
        if c0 + csz > n_classes:
            l2 = jnp.where(col < n_classes - c0, l2,
                           jnp.bfloat16(-jnp.inf))
        cmax = jnp.max(l2, axis=1, keepdims=True).astype(jnp.float32)
        mn = jnp.maximum(m, cmax)
        e = jnp.exp2(l2 - mn.astype(jnp.bfloat16))
        csum = jnp.sum(e, axis=1, keepdims=True).astype(jnp.float32)
        s = s * jnp.exp2(m - mn) + csum
        m = mn
        tg = tg + jnp.sum(
            jnp.where(col == (t - c0), l2, jnp.bfloat16(0)),
            axis=1, keepdims=True).astype(jnp.float32)
    part = (jnp.sum(m + jnp.log2(s) - tg) * (_LN2 / batch)).reshape(1, 1)

    @pl.when(i == 0)
    def _init():
        o_ref[...] = jnp.zeros((1, 1), jnp.float32)

    o_ref[...] += part


def kernel(inputs, targets, lut_ccc, lut_icc):
    b, f = inputs.shape
    n_classes = lut_icc.shape[0]
    bt = _BT if b % _BT == 0 else b
    chunks = []
    c0 = 0
    while c0 < n_classes:
        csz = min(_CT, ((n_classes - c0 + 7) // 8) * 8)
        chunks.append((c0, csz))
        c0 += csz
    cp = c0
    lut_pad = lut_icc if cp == n_classes else jnp.pad(
        lut_icc, ((0, cp - n_classes), (0, 0)))
    x16 = (inputs * (_SCALE * _LOG2E)).astype(jnp.bfloat16)
    lut16 = lut_pad.astype(jnp.bfloat16)
    ones = jnp.ones((chunks[0][1], 128), jnp.bfloat16)
    t2 = targets.reshape(b, 1)
    out = pl.pallas_call(
        lambda xr, tr, lr, onr, orf: _ce_kernel(
            xr, tr, lr, onr, orf, chunks=chunks, bt=bt, batch=b, n_classes=n_classes),
        grid=(b // bt,),
        in_specs=[
            pl.BlockSpec((bt, f), lambda i: (i, 0)),
            pl.BlockSpec((bt, 1), lambda i: (i, 0)),
            pl.BlockSpec((cp, f), lambda i: (0, 0)),
            pl.BlockSpec((chunks[0][1], 128), lambda i: (0, 0)),
        ],
        out_specs=pl.BlockSpec((1, 1), lambda i: (0, 0)),
        out_shape=jax.ShapeDtypeStruct((1, 1), jnp.float32),
    )(x16, t2, lut16, ones)
    loss = out[0, 0]
    return (loss, lut_icc, lut_icc)
